# Initial kernel scaffold; baseline (speedup 1.0000x reference)
import functools

import jax
import jax.numpy as jnp
from jax import lax
from jax.experimental import pallas as pl
from jax.experimental.pallas import tpu as pltpu
from jax.experimental.pallas import tpu_sc as plsc

VOCAB = 457
EMB = 64
BATCH = 16384
SEQ_LEN = 200
N = BATCH * SEQ_LEN

_info = plsc.get_sparse_core_info()
NC = _info.num_cores
NS = _info.num_subcores
NW = NC * NS
SUB = 128
NSUB = 4
CHUNK = SUB * NSUB
PER_W = N // NW
N_ITERS = PER_W // CHUNK

assert N % (NW * CHUNK) == 0


def _sc_lookup(seq_idx, exp_idx, table_seq, table_exp):
    mesh = plsc.VectorSubcoreMesh(core_axis_name="c", subcore_axis_name="s")

    @functools.partial(
        pl.kernel,
        mesh=mesh,
        out_type=(
            jax.ShapeDtypeStruct((N, EMB), jnp.float32),
            jax.ShapeDtypeStruct((N, EMB), jnp.float32),
        ),
        scratch_types=[
            pltpu.VMEM((NSUB, SUB), jnp.int32),
            pltpu.VMEM((CHUNK, EMB), jnp.float32),
            pltpu.SemaphoreType.DMA,
        ],
    )
    def k(seq_hbm, exp_hbm, tseq_hbm, texp_hbm, out_seq, out_exp,
          idx_v, rows_v, sem):
        wid = lax.axis_index("s") * NC + lax.axis_index("c")
        base_irow = wid * (PER_W // SUB)
        base_orow = wid * PER_W

        def run_table(idx_hbm, tab_hbm, out_hbm):
            def one(g, carry):
                irow = base_irow + g * NSUB
                orow = base_orow + g * CHUNK
                pltpu.sync_copy(idx_hbm.at[pl.ds(irow, NSUB)], idx_v)
                copies = [
                    pltpu.async_copy(
                        tab_hbm.at[idx_v.at[j]],
                        rows_v.at[pl.ds(j * SUB, SUB)],
                        sem,
                    )
                    for j in range(NSUB)
                ]
                for c in copies:
                    c.wait()
                pltpu.sync_copy(rows_v, out_hbm.at[pl.ds(orow, CHUNK)])
                return carry

            lax.fori_loop(0, N_ITERS, one, 0)

        run_table(seq_hbm, tseq_hbm, out_seq)
        run_table(exp_hbm, texp_hbm, out_exp)

    return k(seq_idx, exp_idx, table_seq, table_exp)


def kernel(seqs, exps, table_seq, table_exp):
    seq_idx = seqs.astype(jnp.int32).reshape(N // SUB, SUB)
    exp_idx = exps.astype(jnp.int32).reshape(N // SUB, SUB)
    out_seq, out_exp = _sc_lookup(seq_idx, exp_idx, table_seq, table_exp)
    return (
        out_seq.reshape(BATCH, SEQ_LEN, EMB),
        out_exp.reshape(BATCH, SEQ_LEN, EMB),
    )


# speedup vs baseline: 3.8108x; 3.8108x over previous
import functools

import jax
import jax.numpy as jnp
from jax import lax
from jax.experimental import pallas as pl
from jax.experimental.pallas import tpu as pltpu
from jax.experimental.pallas import tpu_sc as plsc

VOCAB = 457
EMB = 64
BATCH = 16384
SEQ_LEN = 200
N = BATCH * SEQ_LEN

_info = plsc.get_sparse_core_info()
NC = _info.num_cores
NS = _info.num_subcores
NW = NC * NS
SUB = 128
NSUB = 4
CHUNK = SUB * NSUB
PER_W = N // NW
N_ITERS = PER_W // CHUNK

assert N % (NW * CHUNK) == 0


def _sc_lookup(seq_idx, exp_idx, table_seq, table_exp):
    mesh = plsc.VectorSubcoreMesh(core_axis_name="c", subcore_axis_name="s")

    @functools.partial(
        pl.kernel,
        mesh=mesh,
        out_type=(
            jax.ShapeDtypeStruct((N, EMB), jnp.float32),
            jax.ShapeDtypeStruct((N, EMB), jnp.float32),
        ),
        scratch_types=[
            pltpu.VMEM((NSUB, SUB), jnp.int32),
            pltpu.VMEM((CHUNK, EMB), jnp.float32),
            pltpu.SemaphoreType.DMA,
        ],
        compiler_params=pltpu.CompilerParams(use_tc_tiling_on_sc=False),
    )
    def k(seq_hbm, exp_hbm, tseq_hbm, texp_hbm, out_seq, out_exp,
          idx_v, rows_v, sem):
        wid = lax.axis_index("s") * NC + lax.axis_index("c")
        base_irow = wid * (PER_W // SUB)
        base_orow = wid * PER_W

        def run_table(idx_hbm, tab_hbm, out_hbm):
            def one(g, carry):
                irow = base_irow + g * NSUB
                orow = base_orow + g * CHUNK
                pltpu.sync_copy(idx_hbm.at[pl.ds(irow, NSUB)], idx_v)
                copies = [
                    pltpu.async_copy(
                        tab_hbm.at[idx_v.at[j]],
                        rows_v.at[pl.ds(j * SUB, SUB)],
                        sem,
                    )
                    for j in range(NSUB)
                ]
                for c in copies:
                    c.wait()
                pltpu.sync_copy(rows_v, out_hbm.at[pl.ds(orow, CHUNK)])
                return carry

            lax.fori_loop(0, N_ITERS, one, 0)

        run_table(seq_hbm, tseq_hbm, out_seq)
        run_table(exp_hbm, texp_hbm, out_exp)

    return k(seq_idx, exp_idx, table_seq, table_exp)


def kernel(seqs, exps, table_seq, table_exp):
    seq_idx = seqs.astype(jnp.int32).reshape(N // SUB, SUB)
    exp_idx = exps.astype(jnp.int32).reshape(N // SUB, SUB)
    out_seq, out_exp = _sc_lookup(seq_idx, exp_idx, table_seq, table_exp)
    return (
        out_seq.reshape(BATCH, SEQ_LEN, EMB),
        out_exp.reshape(BATCH, SEQ_LEN, EMB),
    )
